# baseline (speedup 1.0000x reference)
import jax
import jax.numpy as jnp
from jax.experimental import pallas as pl
from jax.experimental.pallas import tpu as pltpu

_BLK = 2048
_DSPLIT = 2
_E = 64
_HALF = 32
_K = 8


def _gate_kernel(x_ref, w_ref, b_ref, ow_ref, oi_ref, acc_ref):
    j = pl.program_id(1)
    part = jax.lax.dot_general(
        x_ref[...], w_ref[...], (((1,), (1,)), ((), ())),
        preferred_element_type=jnp.float32)

    @pl.when(j == 0)
    def _():
        acc_ref[...] = part

    @pl.when(j == _DSPLIT - 1)
    def _():
        logits = acc_ref[...] + part + b_ref[...]
        lt = jax.lax.transpose(logits, (1, 0))
        R = lt.shape[1]
        m = jnp.max(lt, axis=0, keepdims=True)
        e = jnp.exp(lt - m)
        row = jax.lax.broadcasted_iota(jnp.int32, (_HALF, R), 0)
        neg = jnp.float32(-jnp.inf)

        def top2sum(h):
            m1 = jnp.max(h, axis=0, keepdims=True)
            fi = jnp.min(jnp.where(h == m1, row, _HALF), axis=0, keepdims=True)
            m2 = jnp.max(jnp.where(row == fi, neg, h), axis=0, keepdims=True)
            return m1 + m2

        e0 = e[:_HALF]
        e1 = e[_HALF:]
        g0 = top2sum(e0) >= top2sum(e1)
        s = jnp.where(g0, e0, e1)
        base = jnp.where(g0, 0, _HALF)

        ws = []
        idxs = []
        for _ in range(_K):
            mk = jnp.max(s, axis=0, keepdims=True)
            fi = jnp.min(jnp.where(s == mk, row, _HALF), axis=0, keepdims=True)
            ws.append(mk)
            idxs.append(fi + base)
            s = jnp.where(row == fi, neg, s)
        wmat = jnp.concatenate(ws, axis=0)
        imat = jnp.concatenate(idxs, axis=0)
        wsum = jnp.sum(wmat, axis=0, keepdims=True)
        ow_ref[...] = wmat / jnp.maximum(wsum, 1e-9)
        oi_ref[...] = imat


def kernel(x, W, bias):
    Tloc, dim = x.shape
    e = W.shape[0]
    b2 = bias.reshape(1, e)
    dchunk = dim // _DSPLIT
    grid = (Tloc // _BLK, _DSPLIT)
    ow, oi = pl.pallas_call(
        _gate_kernel,
        grid=grid,
        in_specs=[
            pl.BlockSpec((_BLK, dchunk), lambda i, j: (i, j)),
            pl.BlockSpec((e, dchunk), lambda i, j: (0, j)),
            pl.BlockSpec((1, e), lambda i, j: (0, 0)),
        ],
        out_specs=[
            pl.BlockSpec((_K, _BLK), lambda i, j: (0, i)),
            pl.BlockSpec((_K, _BLK), lambda i, j: (0, i)),
        ],
        out_shape=[
            jax.ShapeDtypeStruct((_K, Tloc), jnp.float32),
            jax.ShapeDtypeStruct((_K, Tloc), jnp.int32),
        ],
        scratch_shapes=[pltpu.VMEM((_BLK, _E), jnp.float32)],
    )(x, W, b2)
    return (ow.T.astype(x.dtype), oi.T)


# speedup vs baseline: 1.1599x; 1.1599x over previous
import jax
import jax.numpy as jnp
from jax.experimental import pallas as pl
from jax.experimental.pallas import tpu as pltpu

_BLK = 2048
_E = 64
_HALF = 32
_K = 8


def _gate_kernel(xa_ref, xb_ref, w_ref, b_ref, ow_ref, oi_ref):
    b = b_ref[...]
    Wm = w_ref[...]
    la = jax.lax.dot_general(
        xa_ref[...], Wm, (((1,), (1,)), ((), ())),
        preferred_element_type=jnp.float32) + b
    lb = jax.lax.dot_general(
        xb_ref[...], Wm, (((1,), (1,)), ((), ())),
        preferred_element_type=jnp.float32) + b
    lt = jnp.concatenate(
        [jax.lax.transpose(la, (1, 0)),
         jax.lax.transpose(lb, (1, 0))], axis=1)
    R = lt.shape[1]
    m = jnp.max(lt, axis=0, keepdims=True)
    e = jnp.exp(lt - m)
    row = jax.lax.broadcasted_iota(jnp.int32, (_HALF, R), 0)
    neg = jnp.float32(-jnp.inf)

    def top2sum(h):
        m1 = jnp.max(h, axis=0, keepdims=True)
        fi = jnp.min(jnp.where(h == m1, row, _HALF), axis=0, keepdims=True)
        m2 = jnp.max(jnp.where(row == fi, neg, h), axis=0, keepdims=True)
        return m1 + m2

    e0 = e[:_HALF]
    e1 = e[_HALF:]
    rep0 = top2sum(e0)
    rep1 = top2sum(e1)
    g0 = rep0 >= rep1
    s = jnp.where(g0, e0, e1)
    base = jnp.where(g0, 0, _HALF)

    ws = []
    idxs = []
    for _ in range(_K):
        mk = jnp.max(s, axis=0, keepdims=True)
        fi = jnp.min(jnp.where(s == mk, row, _HALF), axis=0, keepdims=True)
        ws.append(mk)
        idxs.append(fi + base)
        s = jnp.where(row == fi, neg, s)
    wmat = jnp.concatenate(ws, axis=0)
    imat = jnp.concatenate(idxs, axis=0)
    wsum = jnp.sum(wmat, axis=0, keepdims=True)
    ow_ref[...] = wmat / jnp.maximum(wsum, 1e-9)
    oi_ref[...] = imat


def kernel(x, W, bias):
    Tloc, dim = x.shape
    e = W.shape[0]
    b2 = bias.reshape(1, e)
    grid = (Tloc // _BLK,)
    ow, oi = pl.pallas_call(
        _gate_kernel,
        grid=grid,
        in_specs=[
            pl.BlockSpec((_BLK // 2, dim), lambda i: (2 * i, 0)),
            pl.BlockSpec((_BLK // 2, dim), lambda i: (2 * i + 1, 0)),
            pl.BlockSpec((e, dim), lambda i: (0, 0)),
            pl.BlockSpec((1, e), lambda i: (0, 0)),
        ],
        out_specs=[
            pl.BlockSpec((_K, _BLK), lambda i: (0, i)),
            pl.BlockSpec((_K, _BLK), lambda i: (0, i)),
        ],
        out_shape=[
            jax.ShapeDtypeStruct((_K, Tloc), jnp.float32),
            jax.ShapeDtypeStruct((_K, Tloc), jnp.int32),
        ],
    )(x, x, W, b2)
    return (ow.T.astype(x.dtype), oi.T)
